# initial kernel scaffold (unmeasured)
import jax
import jax.numpy as jnp
from jax import lax
from jax.experimental import pallas as pl
from jax.experimental.pallas import tpu as pltpu


def kernel(
    x,
):
    def body(*refs):
        pass

    out_shape = jax.ShapeDtypeStruct(..., jnp.float32)
    return pl.pallas_call(body, out_shape=out_shape)(...)


# baseline (device time: 585323 ns/iter reference)
import jax
import jax.numpy as jnp
from jax import lax
from jax.experimental import pallas as pl
from jax.experimental.pallas import tpu as pltpu

N_DEV = 4


def kernel(x):
    m_per, n = x.shape
    chunk = m_per // N_DEV

    def body(
        x_ref,
        out_ref,
        x_stage,
        send_buf,
        rs_recv,
        rs_send_sems,
        rs_recv_sems,
        ag_send_sems,
        ag_recv_sems,
        local_sem,
        store_sem,
    ):
        my = lax.axis_index("i")
        left = (my - 1) % N_DEV
        right = (my + 1) % N_DEV

        barrier = pltpu.get_barrier_semaphore()
        for nbr in (left, right):
            pl.semaphore_signal(
                barrier, inc=1,
                device_id=(nbr,), device_id_type=pl.DeviceIdType.MESH,
            )
        pl.semaphore_wait(barrier, 2)

        cp = pltpu.make_async_copy(
            x_ref.at[pl.ds(my * chunk, chunk), :], x_stage, local_sem
        )
        cp.start()
        cp.wait()
        send_buf[...] = x_stage[...].astype(jnp.bfloat16)

        for s in range(N_DEV - 1):
            rdma = pltpu.make_async_remote_copy(
                src_ref=send_buf,
                dst_ref=rs_recv.at[s],
                send_sem=rs_send_sems.at[s],
                recv_sem=rs_recv_sems.at[s],
                device_id=(right,),
                device_id_type=pl.DeviceIdType.MESH,
            )
            rdma.start()
            nxt = (my - s - 1) % N_DEV
            cp = pltpu.make_async_copy(
                x_ref.at[pl.ds(nxt * chunk, chunk), :], x_stage, local_sem
            )
            cp.start()
            rdma.wait()
            cp.wait()
            send_buf[...] = (
                x_stage[...] + rs_recv[s].astype(jnp.float32)
            ).astype(jnp.bfloat16)

        red = (my + 1) % N_DEV
        st = pltpu.make_async_copy(
            send_buf, out_ref.at[pl.ds(red * chunk, chunk), :], store_sem
        )
        st.start()

        for h in range(N_DEV - 1):
            send_c = (my + 1 - h) % N_DEV
            if h == 0:
                src = send_buf
            else:
                src = out_ref.at[pl.ds(send_c * chunk, chunk), :]
            rdma = pltpu.make_async_remote_copy(
                src_ref=src,
                dst_ref=out_ref.at[pl.ds(send_c * chunk, chunk), :],
                send_sem=ag_send_sems.at[h],
                recv_sem=ag_recv_sems.at[h],
                device_id=(right,),
                device_id_type=pl.DeviceIdType.MESH,
            )
            rdma.start()
            rdma.wait()

        st.wait()

    return pl.pallas_call(
        body,
        out_shape=jax.ShapeDtypeStruct((m_per, n), jnp.bfloat16),
        in_specs=[pl.BlockSpec(memory_space=pl.ANY)],
        out_specs=pl.BlockSpec(memory_space=pl.ANY),
        scratch_shapes=[
            pltpu.VMEM((chunk, n), jnp.float32),
            pltpu.VMEM((chunk, n), jnp.bfloat16),
            pltpu.VMEM((N_DEV - 1, chunk, n), jnp.bfloat16),
            pltpu.SemaphoreType.DMA((N_DEV - 1,)),
            pltpu.SemaphoreType.DMA((N_DEV - 1,)),
            pltpu.SemaphoreType.DMA((N_DEV - 1,)),
            pltpu.SemaphoreType.DMA((N_DEV - 1,)),
            pltpu.SemaphoreType.DMA,
            pltpu.SemaphoreType.DMA,
        ],
        compiler_params=pltpu.CompilerParams(
            collective_id=0,
            vmem_limit_bytes=60 * 1024 * 1024,
        ),
    )(x)


# device time: 315496 ns/iter; 1.8552x vs baseline; 1.8552x over previous
import jax
import jax.numpy as jnp
from jax import lax
from jax.experimental import pallas as pl
from jax.experimental.pallas import tpu as pltpu

N_DEV = 4


def kernel(x):
    m_per, n = x.shape
    half = m_per // 2
    chunk = half // N_DEV

    def body(
        x_ref,
        out_ref,
        cw_stage,
        ccw_stage,
        cw_send,
        ccw_send,
        cw_rs_recv,
        ccw_rs_recv,
        cw_rs_send_sems,
        cw_rs_recv_sems,
        ccw_rs_send_sems,
        ccw_rs_recv_sems,
        cw_ag_send_sems,
        cw_ag_recv_sems,
        ccw_ag_send_sems,
        ccw_ag_recv_sems,
        cw_local_sem,
        ccw_local_sem,
        cw_store_sem,
        ccw_store_sem,
    ):
        my = lax.axis_index("i")
        left = (my - 1) % N_DEV
        right = (my + 1) % N_DEV

        def cw_rows(c):
            return pl.ds(c * chunk, chunk)

        def ccw_rows(c):
            return pl.ds(half + c * chunk, chunk)

        barrier = pltpu.get_barrier_semaphore()
        for nbr in (left, right):
            pl.semaphore_signal(
                barrier, inc=1,
                device_id=(nbr,), device_id_type=pl.DeviceIdType.MESH,
            )
        pl.semaphore_wait(barrier, 2)

        cp_cw = pltpu.make_async_copy(
            x_ref.at[cw_rows(my), :], cw_stage, cw_local_sem
        )
        cp_ccw = pltpu.make_async_copy(
            x_ref.at[ccw_rows(my), :], ccw_stage, ccw_local_sem
        )
        cp_cw.start()
        cp_ccw.start()
        cp_cw.wait()
        cw_send[...] = cw_stage[...].astype(jnp.bfloat16)
        cp_ccw.wait()
        ccw_send[...] = ccw_stage[...].astype(jnp.bfloat16)

        for s in range(N_DEV - 1):
            rdma_cw = pltpu.make_async_remote_copy(
                src_ref=cw_send,
                dst_ref=cw_rs_recv.at[s],
                send_sem=cw_rs_send_sems.at[s],
                recv_sem=cw_rs_recv_sems.at[s],
                device_id=(right,),
                device_id_type=pl.DeviceIdType.MESH,
            )
            rdma_ccw = pltpu.make_async_remote_copy(
                src_ref=ccw_send,
                dst_ref=ccw_rs_recv.at[s],
                send_sem=ccw_rs_send_sems.at[s],
                recv_sem=ccw_rs_recv_sems.at[s],
                device_id=(left,),
                device_id_type=pl.DeviceIdType.MESH,
            )
            rdma_cw.start()
            rdma_ccw.start()
            cw_nxt = (my - s - 1) % N_DEV
            ccw_nxt = (my + s + 1) % N_DEV
            cp_cw = pltpu.make_async_copy(
                x_ref.at[cw_rows(cw_nxt), :], cw_stage, cw_local_sem
            )
            cp_ccw = pltpu.make_async_copy(
                x_ref.at[ccw_rows(ccw_nxt), :], ccw_stage, ccw_local_sem
            )
            cp_cw.start()
            cp_ccw.start()
            rdma_cw.wait()
            cp_cw.wait()
            cw_send[...] = (
                cw_stage[...] + cw_rs_recv[s].astype(jnp.float32)
            ).astype(jnp.bfloat16)
            rdma_ccw.wait()
            cp_ccw.wait()
            ccw_send[...] = (
                ccw_stage[...] + ccw_rs_recv[s].astype(jnp.float32)
            ).astype(jnp.bfloat16)

        cw_red = (my + 1) % N_DEV
        ccw_red = (my - 1) % N_DEV
        st_cw = pltpu.make_async_copy(
            cw_send, out_ref.at[cw_rows(cw_red), :], cw_store_sem
        )
        st_ccw = pltpu.make_async_copy(
            ccw_send, out_ref.at[ccw_rows(ccw_red), :], ccw_store_sem
        )
        st_cw.start()
        st_ccw.start()

        for h in range(N_DEV - 1):
            cw_c = (my + 1 - h) % N_DEV
            ccw_c = (my - 1 + h) % N_DEV
            cw_src = cw_send if h == 0 else out_ref.at[cw_rows(cw_c), :]
            ccw_src = ccw_send if h == 0 else out_ref.at[ccw_rows(ccw_c), :]
            rdma_cw = pltpu.make_async_remote_copy(
                src_ref=cw_src,
                dst_ref=out_ref.at[cw_rows(cw_c), :],
                send_sem=cw_ag_send_sems.at[h],
                recv_sem=cw_ag_recv_sems.at[h],
                device_id=(right,),
                device_id_type=pl.DeviceIdType.MESH,
            )
            rdma_ccw = pltpu.make_async_remote_copy(
                src_ref=ccw_src,
                dst_ref=out_ref.at[ccw_rows(ccw_c), :],
                send_sem=ccw_ag_send_sems.at[h],
                recv_sem=ccw_ag_recv_sems.at[h],
                device_id=(left,),
                device_id_type=pl.DeviceIdType.MESH,
            )
            rdma_cw.start()
            rdma_ccw.start()
            rdma_cw.wait()
            rdma_ccw.wait()

        st_cw.wait()
        st_ccw.wait()

    return pl.pallas_call(
        body,
        out_shape=jax.ShapeDtypeStruct((m_per, n), jnp.bfloat16),
        in_specs=[pl.BlockSpec(memory_space=pl.ANY)],
        out_specs=pl.BlockSpec(memory_space=pl.ANY),
        scratch_shapes=[
            pltpu.VMEM((chunk, n), jnp.float32),
            pltpu.VMEM((chunk, n), jnp.float32),
            pltpu.VMEM((chunk, n), jnp.bfloat16),
            pltpu.VMEM((chunk, n), jnp.bfloat16),
            pltpu.VMEM((N_DEV - 1, chunk, n), jnp.bfloat16),
            pltpu.VMEM((N_DEV - 1, chunk, n), jnp.bfloat16),
            pltpu.SemaphoreType.DMA((N_DEV - 1,)),
            pltpu.SemaphoreType.DMA((N_DEV - 1,)),
            pltpu.SemaphoreType.DMA((N_DEV - 1,)),
            pltpu.SemaphoreType.DMA((N_DEV - 1,)),
            pltpu.SemaphoreType.DMA((N_DEV - 1,)),
            pltpu.SemaphoreType.DMA((N_DEV - 1,)),
            pltpu.SemaphoreType.DMA((N_DEV - 1,)),
            pltpu.SemaphoreType.DMA((N_DEV - 1,)),
            pltpu.SemaphoreType.DMA,
            pltpu.SemaphoreType.DMA,
            pltpu.SemaphoreType.DMA,
            pltpu.SemaphoreType.DMA,
        ],
        compiler_params=pltpu.CompilerParams(
            collective_id=0,
            vmem_limit_bytes=60 * 1024 * 1024,
        ),
    )(x)


# device time: 300183 ns/iter; 1.9499x vs baseline; 1.0510x over previous
import jax
import jax.numpy as jnp
from jax import lax
from jax.experimental import pallas as pl
from jax.experimental.pallas import tpu as pltpu

N_DEV = 4
SUB = 2


def kernel(x):
    m_per, n = x.shape
    half = m_per // 2
    chunk = half // N_DEV
    sub = chunk // SUB

    def body(
        x_ref,
        out_ref,
        cw_stage, ccw_stage,
        cw_send, ccw_send,
        cw_recv, ccw_recv,
        cw_rs_ssem, cw_rs_rsem,
        ccw_rs_ssem, ccw_rs_rsem,
        cw_ag_ssem, cw_ag_rsem,
        ccw_ag_ssem, ccw_ag_rsem,
        cw_stg_sems, ccw_stg_sems,
        cw_store_sem, ccw_store_sem,
    ):
        my = lax.axis_index("i")
        left = (my - 1) % N_DEV
        right = (my + 1) % N_DEV
        f32 = jnp.float32
        bf16 = jnp.bfloat16

        dirs = [
            dict(nbr=right, sgn=-1, base=0, stage=cw_stage, send=cw_send,
                 recv=cw_recv, rs_ssem=cw_rs_ssem, rs_rsem=cw_rs_rsem,
                 ag_ssem=cw_ag_ssem, ag_rsem=cw_ag_rsem, stg=cw_stg_sems,
                 store=cw_store_sem),
            dict(nbr=left, sgn=+1, base=half, stage=ccw_stage, send=ccw_send,
                 recv=ccw_recv, rs_ssem=ccw_rs_ssem, rs_rsem=ccw_rs_rsem,
                 ag_ssem=ccw_ag_ssem, ag_rsem=ccw_ag_rsem, stg=ccw_stg_sems,
                 store=ccw_store_sem),
        ]

        def rows(d, c):
            return pl.ds(d["base"] + c * chunk, chunk)

        def subrows(d, c, k):
            return pl.ds(d["base"] + c * chunk + k * sub, sub)

        def sb(k):
            return pl.ds(k * sub, sub)

        def stage_start(d, c, k):
            cp = pltpu.make_async_copy(
                x_ref.at[subrows(d, c, k), :],
                d["stage"].at[sb(k), :],
                d["stg"].at[k],
            )
            cp.start()
            return cp

        def rs_rdma(d, s, k):
            i = s * SUB + k
            return pltpu.make_async_remote_copy(
                src_ref=d["send"].at[sb(k), :],
                dst_ref=d["recv"].at[s, sb(k), :],
                send_sem=d["rs_ssem"].at[i],
                recv_sem=d["rs_rsem"].at[i],
                device_id=(d["nbr"],),
                device_id_type=pl.DeviceIdType.MESH,
            )

        def ag_rdma(d, h, k):
            c = (my + d["sgn"] * (h - 1)) % N_DEV
            i = h * SUB + k
            src = d["send"].at[sb(k), :] if h == 0 else out_ref.at[subrows(d, c, k), :]
            return pltpu.make_async_remote_copy(
                src_ref=src,
                dst_ref=out_ref.at[subrows(d, c, k), :],
                send_sem=d["ag_ssem"].at[i],
                recv_sem=d["ag_rsem"].at[i],
                device_id=(d["nbr"],),
                device_id_type=pl.DeviceIdType.MESH,
            )

        barrier = pltpu.get_barrier_semaphore()
        for nbr in (left, right):
            pl.semaphore_signal(
                barrier, inc=1,
                device_id=(nbr,), device_id_type=pl.DeviceIdType.MESH,
            )
        pl.semaphore_wait(barrier, 2)

        for d in dirs:
            d["cp"] = [stage_start(d, my, k) for k in range(SUB)]
        for k in range(SUB):
            for d in dirs:
                d["cp"][k].wait()
                d["send"][sb(k), :] = d["stage"][sb(k), :].astype(bf16)
                rs_rdma(d, 0, k).start()
        for d in dirs:
            d["cp"] = [
                stage_start(d, (my + d["sgn"]) % N_DEV, k) for k in range(SUB)
            ]

        for s in range(N_DEV - 1):
            for k in range(SUB):
                for d in dirs:
                    rs_rdma(d, s, k).wait()
                    d["cp"][k].wait()
                    d["send"][sb(k), :] = (
                        d["stage"][sb(k), :]
                        + d["recv"][s, sb(k), :].astype(f32)
                    ).astype(bf16)
                    if s < N_DEV - 2:
                        rs_rdma(d, s + 1, k).start()
                        d["cp"][k] = stage_start(
                            d, (my + d["sgn"] * (s + 2)) % N_DEV, k
                        )
                    else:
                        ag_rdma(d, 0, k).start()

        for d in dirs:
            red = (my - d["sgn"]) % N_DEV
            d["st"] = pltpu.make_async_copy(
                d["send"], out_ref.at[rows(d, red), :], d["store"]
            )
            d["st"].start()

        for h in range(1, N_DEV - 1):
            for k in range(SUB):
                for d in dirs:
                    ag_rdma(d, h - 1, k).wait()
                    ag_rdma(d, h, k).start()
        for k in range(SUB):
            for d in dirs:
                ag_rdma(d, N_DEV - 2, k).wait()
        for d in dirs:
            d["st"].wait()

    nsem = (N_DEV - 1) * SUB
    return pl.pallas_call(
        body,
        out_shape=jax.ShapeDtypeStruct((m_per, n), jnp.bfloat16),
        in_specs=[pl.BlockSpec(memory_space=pl.ANY)],
        out_specs=pl.BlockSpec(memory_space=pl.ANY),
        scratch_shapes=[
            pltpu.VMEM((chunk, n), jnp.float32),
            pltpu.VMEM((chunk, n), jnp.float32),
            pltpu.VMEM((chunk, n), jnp.bfloat16),
            pltpu.VMEM((chunk, n), jnp.bfloat16),
            pltpu.VMEM((N_DEV - 1, chunk, n), jnp.bfloat16),
            pltpu.VMEM((N_DEV - 1, chunk, n), jnp.bfloat16),
            pltpu.SemaphoreType.DMA((nsem,)),
            pltpu.SemaphoreType.DMA((nsem,)),
            pltpu.SemaphoreType.DMA((nsem,)),
            pltpu.SemaphoreType.DMA((nsem,)),
            pltpu.SemaphoreType.DMA((nsem,)),
            pltpu.SemaphoreType.DMA((nsem,)),
            pltpu.SemaphoreType.DMA((nsem,)),
            pltpu.SemaphoreType.DMA((nsem,)),
            pltpu.SemaphoreType.DMA((SUB,)),
            pltpu.SemaphoreType.DMA((SUB,)),
            pltpu.SemaphoreType.DMA,
            pltpu.SemaphoreType.DMA,
        ],
        compiler_params=pltpu.CompilerParams(
            collective_id=0,
            vmem_limit_bytes=60 * 1024 * 1024,
        ),
    )(x)


# device time: 298869 ns/iter; 1.9585x vs baseline; 1.0044x over previous
import jax
import jax.numpy as jnp
from jax import lax
from jax.experimental import pallas as pl
from jax.experimental.pallas import tpu as pltpu

N_DEV = 4
SUB = 2


def kernel(x):
    m_per, n = x.shape
    half = m_per // 2
    chunk = half // N_DEV
    sub = chunk // SUB

    def body(
        x_ref,
        out_ref,
        cw_stage, ccw_stage,
        cw_send, ccw_send,
        cw_recv, ccw_recv,
        cw_rs_ssem, cw_rs_rsem,
        ccw_rs_ssem, ccw_rs_rsem,
        cw_ag_ssem, cw_ag_rsem,
        ccw_ag_ssem, ccw_ag_rsem,
        cw_stg_sems, ccw_stg_sems,
        cw_store_sem, ccw_store_sem,
    ):
        my = lax.axis_index("i")
        left = (my - 1) % N_DEV
        right = (my + 1) % N_DEV
        f32 = jnp.float32
        bf16 = jnp.bfloat16

        dirs = [
            dict(nbr=right, sgn=-1, base=0, stage=cw_stage, send=cw_send,
                 recv=cw_recv, rs_ssem=cw_rs_ssem, rs_rsem=cw_rs_rsem,
                 ag_ssem=cw_ag_ssem, ag_rsem=cw_ag_rsem, stg=cw_stg_sems,
                 store=cw_store_sem),
            dict(nbr=left, sgn=+1, base=half, stage=ccw_stage, send=ccw_send,
                 recv=ccw_recv, rs_ssem=ccw_rs_ssem, rs_rsem=ccw_rs_rsem,
                 ag_ssem=ccw_ag_ssem, ag_rsem=ccw_ag_rsem, stg=ccw_stg_sems,
                 store=ccw_store_sem),
        ]

        def rows(d, c):
            return pl.ds(d["base"] + c * chunk, chunk)

        def subrows(d, c, k):
            return pl.ds(d["base"] + c * chunk + k * sub, sub)

        def sb(k):
            return pl.ds(k * sub, sub)

        def stage_start(d, c, k):
            cp = pltpu.make_async_copy(
                x_ref.at[subrows(d, c, k), :],
                d["stage"].at[sb(k), :],
                d["stg"].at[k],
            )
            cp.start()
            return cp

        def rs_rdma(d, s, k):
            i = s * SUB + k
            return pltpu.make_async_remote_copy(
                src_ref=d["send"].at[sb(k), :],
                dst_ref=d["recv"].at[s, sb(k), :],
                send_sem=d["rs_ssem"].at[i],
                recv_sem=d["rs_rsem"].at[i],
                device_id=(d["nbr"],),
                device_id_type=pl.DeviceIdType.MESH,
            )

        def ag_rdma(d, h, k):
            c = (my + d["sgn"] * (h - 1)) % N_DEV
            i = h * SUB + k
            src = d["send"].at[sb(k), :] if h == 0 else out_ref.at[subrows(d, c, k), :]
            return pltpu.make_async_remote_copy(
                src_ref=src,
                dst_ref=out_ref.at[subrows(d, c, k), :],
                send_sem=d["ag_ssem"].at[i],
                recv_sem=d["ag_rsem"].at[i],
                device_id=(d["nbr"],),
                device_id_type=pl.DeviceIdType.MESH,
            )

        for d in dirs:
            d["cp"] = [stage_start(d, my, k) for k in range(SUB)]

        barrier = pltpu.get_barrier_semaphore()
        for nbr in (left, right):
            pl.semaphore_signal(
                barrier, inc=1,
                device_id=(nbr,), device_id_type=pl.DeviceIdType.MESH,
            )
        pl.semaphore_wait(barrier, 2)

        for k in range(SUB):
            for d in dirs:
                d["cp"][k].wait()
                d["send"][sb(k), :] = d["stage"][sb(k), :].astype(bf16)
                rs_rdma(d, 0, k).start()
        for d in dirs:
            d["cp"] = [
                stage_start(d, (my + d["sgn"]) % N_DEV, k) for k in range(SUB)
            ]

        for s in range(N_DEV - 1):
            for k in range(SUB):
                for d in dirs:
                    rs_rdma(d, s, k).wait()
                    d["cp"][k].wait()
                    d["send"][sb(k), :] = (
                        d["stage"][sb(k), :]
                        + d["recv"][s, sb(k), :].astype(f32)
                    ).astype(bf16)
                    if s < N_DEV - 2:
                        rs_rdma(d, s + 1, k).start()
                        d["cp"][k] = stage_start(
                            d, (my + d["sgn"] * (s + 2)) % N_DEV, k
                        )
                    else:
                        ag_rdma(d, 0, k).start()

        for d in dirs:
            red = (my - d["sgn"]) % N_DEV
            d["st"] = pltpu.make_async_copy(
                d["send"], out_ref.at[rows(d, red), :], d["store"]
            )
            d["st"].start()

        for h in range(1, N_DEV - 1):
            for k in range(SUB):
                for d in dirs:
                    ag_rdma(d, h - 1, k).wait()
                    ag_rdma(d, h, k).start()
        for k in range(SUB):
            for d in dirs:
                ag_rdma(d, N_DEV - 2, k).wait()
        for d in dirs:
            d["st"].wait()

    nsem = (N_DEV - 1) * SUB
    return pl.pallas_call(
        body,
        out_shape=jax.ShapeDtypeStruct((m_per, n), jnp.bfloat16),
        in_specs=[pl.BlockSpec(memory_space=pl.ANY)],
        out_specs=pl.BlockSpec(memory_space=pl.ANY),
        scratch_shapes=[
            pltpu.VMEM((chunk, n), jnp.float32),
            pltpu.VMEM((chunk, n), jnp.float32),
            pltpu.VMEM((chunk, n), jnp.bfloat16),
            pltpu.VMEM((chunk, n), jnp.bfloat16),
            pltpu.VMEM((N_DEV - 1, chunk, n), jnp.bfloat16),
            pltpu.VMEM((N_DEV - 1, chunk, n), jnp.bfloat16),
            pltpu.SemaphoreType.DMA((nsem,)),
            pltpu.SemaphoreType.DMA((nsem,)),
            pltpu.SemaphoreType.DMA((nsem,)),
            pltpu.SemaphoreType.DMA((nsem,)),
            pltpu.SemaphoreType.DMA((nsem,)),
            pltpu.SemaphoreType.DMA((nsem,)),
            pltpu.SemaphoreType.DMA((nsem,)),
            pltpu.SemaphoreType.DMA((nsem,)),
            pltpu.SemaphoreType.DMA((SUB,)),
            pltpu.SemaphoreType.DMA((SUB,)),
            pltpu.SemaphoreType.DMA,
            pltpu.SemaphoreType.DMA,
        ],
        compiler_params=pltpu.CompilerParams(
            collective_id=0,
            vmem_limit_bytes=60 * 1024 * 1024,
        ),
    )(x)


# device time: 297558 ns/iter; 1.9671x vs baseline; 1.0044x over previous
import jax
import jax.numpy as jnp
from jax import lax
from jax.experimental import pallas as pl
from jax.experimental.pallas import tpu as pltpu

N_DEV = 4
SUB = 4


def kernel(x):
    m_per, n = x.shape
    half = m_per // 2
    chunk = half // N_DEV
    sub = chunk // SUB

    def body(
        x_ref,
        out_ref,
        cw_stage, ccw_stage,
        cw_send, ccw_send,
        cw_recv, ccw_recv,
        cw_ssem, ccw_ssem,
        cw_rs_rsem, ccw_rs_rsem,
        cw_ag_rsem, ccw_ag_rsem,
        cw_stg_sems, ccw_stg_sems,
        cw_store_sem, ccw_store_sem,
    ):
        my = lax.axis_index("i")
        left = (my - 1) % N_DEV
        right = (my + 1) % N_DEV
        f32 = jnp.float32
        bf16 = jnp.bfloat16

        dirs = [
            dict(nbr=right, sgn=-1, base=0, stage=cw_stage, send=cw_send,
                 recv=cw_recv, ssem=cw_ssem, rs_rsem=cw_rs_rsem,
                 ag_rsem=cw_ag_rsem, stg=cw_stg_sems, store=cw_store_sem),
            dict(nbr=left, sgn=+1, base=half, stage=ccw_stage, send=ccw_send,
                 recv=ccw_recv, ssem=ccw_ssem, rs_rsem=ccw_rs_rsem,
                 ag_rsem=ccw_ag_rsem, stg=ccw_stg_sems, store=ccw_store_sem),
        ]

        def rows(d, c):
            return pl.ds(d["base"] + c * chunk, chunk)

        def subrows(d, c, k):
            return pl.ds(d["base"] + c * chunk + k * sub, sub)

        def sb(k):
            return pl.ds(k * sub, sub)

        def stage_start(d, c, k):
            cp = pltpu.make_async_copy(
                x_ref.at[subrows(d, c, k), :],
                d["stage"].at[sb(k), :],
                d["stg"].at[k],
            )
            cp.start()
            return cp

        def rs_rdma(d, s, k):
            return pltpu.make_async_remote_copy(
                src_ref=d["send"].at[sb(k), :],
                dst_ref=d["recv"].at[s, sb(k), :],
                send_sem=d["ssem"].at[k],
                recv_sem=d["rs_rsem"].at[s * SUB + k],
                device_id=(d["nbr"],),
                device_id_type=pl.DeviceIdType.MESH,
            )

        def ag_rdma(d, h, k):
            c = (my + d["sgn"] * (h - 1)) % N_DEV
            src = d["send"].at[sb(k), :] if h == 0 else out_ref.at[subrows(d, c, k), :]
            return pltpu.make_async_remote_copy(
                src_ref=src,
                dst_ref=out_ref.at[subrows(d, c, k), :],
                send_sem=d["ssem"].at[k],
                recv_sem=d["ag_rsem"].at[h * SUB + k],
                device_id=(d["nbr"],),
                device_id_type=pl.DeviceIdType.MESH,
            )

        for d in dirs:
            d["cp"] = [stage_start(d, my, k) for k in range(SUB)]

        barrier = pltpu.get_barrier_semaphore()
        for nbr in (left, right):
            pl.semaphore_signal(
                barrier, inc=1,
                device_id=(nbr,), device_id_type=pl.DeviceIdType.MESH,
            )
        pl.semaphore_wait(barrier, 2)

        for k in range(SUB):
            for d in dirs:
                d["cp"][k].wait()
                d["send"][sb(k), :] = d["stage"][sb(k), :].astype(bf16)
                rs_rdma(d, 0, k).start()
        for d in dirs:
            d["cp"] = [
                stage_start(d, (my + d["sgn"]) % N_DEV, k) for k in range(SUB)
            ]

        for s in range(N_DEV - 1):
            for k in range(SUB):
                for d in dirs:
                    rs_rdma(d, s, k).wait()
                    d["cp"][k].wait()
                    d["send"][sb(k), :] = (
                        d["stage"][sb(k), :]
                        + d["recv"][s, sb(k), :].astype(f32)
                    ).astype(bf16)
                    if s < N_DEV - 2:
                        rs_rdma(d, s + 1, k).start()
                        d["cp"][k] = stage_start(
                            d, (my + d["sgn"] * (s + 2)) % N_DEV, k
                        )
                    else:
                        ag_rdma(d, 0, k).start()

        for d in dirs:
            red = (my - d["sgn"]) % N_DEV
            d["st"] = pltpu.make_async_copy(
                d["send"], out_ref.at[rows(d, red), :], d["store"]
            )
            d["st"].start()

        for h in range(1, N_DEV - 1):
            for k in range(SUB):
                for d in dirs:
                    ag_rdma(d, h - 1, k).wait()
                    ag_rdma(d, h, k).start()
        for k in range(SUB):
            for d in dirs:
                ag_rdma(d, N_DEV - 2, k).wait()
        for d in dirs:
            d["st"].wait()

    nsem = (N_DEV - 1) * SUB
    return pl.pallas_call(
        body,
        out_shape=jax.ShapeDtypeStruct((m_per, n), jnp.bfloat16),
        in_specs=[pl.BlockSpec(memory_space=pl.ANY)],
        out_specs=pl.BlockSpec(memory_space=pl.ANY),
        scratch_shapes=[
            pltpu.VMEM((chunk, n), jnp.float32),
            pltpu.VMEM((chunk, n), jnp.float32),
            pltpu.VMEM((chunk, n), jnp.bfloat16),
            pltpu.VMEM((chunk, n), jnp.bfloat16),
            pltpu.VMEM((N_DEV - 1, chunk, n), jnp.bfloat16),
            pltpu.VMEM((N_DEV - 1, chunk, n), jnp.bfloat16),
            pltpu.SemaphoreType.DMA((SUB,)),
            pltpu.SemaphoreType.DMA((SUB,)),
            pltpu.SemaphoreType.DMA((nsem,)),
            pltpu.SemaphoreType.DMA((nsem,)),
            pltpu.SemaphoreType.DMA((nsem,)),
            pltpu.SemaphoreType.DMA((nsem,)),
            pltpu.SemaphoreType.DMA((SUB,)),
            pltpu.SemaphoreType.DMA((SUB,)),
            pltpu.SemaphoreType.DMA,
            pltpu.SemaphoreType.DMA,
        ],
        compiler_params=pltpu.CompilerParams(
            collective_id=0,
            vmem_limit_bytes=60 * 1024 * 1024,
        ),
    )(x)
